# baseline (speedup 1.0000x reference)
import functools

import jax
import jax.numpy as jnp
from jax import lax
from jax.experimental import pallas as pl
from jax.experimental.pallas import tpu as pltpu
from jax.experimental.pallas import tpu_sc as plsc

G = 4096
D = 128
N = 100000
NB = 8
NC, NS, L = 2, 16, 16
NW = NC * NS
BPW = 3136
CH = 224
NCH = BPW // CH
BPW_LAST = N - (NW - 1) * BPW
PARTIAL_C = BPW_LAST // CH
PARTIAL_LEN = BPW_LAST - PARTIAL_C * CH


def _emb_body(tau_ref, emb_ref, alpha_ref, sigma_ref):
    tau = tau_ref[:, :]
    col = lax.broadcasted_iota(jnp.int32, (1, D), 1)
    half = col >= D // 2
    j = jnp.where(half, col - D // 2, col).astype(jnp.float32)
    freq = jnp.exp(-jnp.log(10000.0) * j / (D // 2))
    phase = jnp.where(half, jnp.float32(jnp.pi / 2), jnp.float32(0.0))
    x = tau * freq + phase
    x2 = x * x
    p = jnp.float32(-1.0 / 39916800.0)
    p = p * x2 + jnp.float32(1.0 / 362880.0)
    p = p * x2 + jnp.float32(-1.0 / 5040.0)
    p = p * x2 + jnp.float32(1.0 / 120.0)
    p = p * x2 + jnp.float32(-1.0 / 6.0)
    emb_ref[:, :] = x * (p * x2 + jnp.float32(1.0))
    alpha_ref[:, :] = 1.0 - tau
    sigma_ref[:, :] = tau


_emb_call = pl.pallas_call(
    _emb_body,
    out_shape=[
        jax.ShapeDtypeStruct((G, D), jnp.float32),
        jax.ShapeDtypeStruct((G, 1), jnp.float32),
        jax.ShapeDtypeStruct((G, 1), jnp.float32),
    ],
)


def _bits_body(parts_ref, bits_ref):
    counts = jnp.sum(parts_ref[:, :], axis=0, keepdims=True)
    row = lax.broadcasted_iota(jnp.int32, (NB, G), 0)
    bits_ref[:, :] = ((counts >> row) & 1).astype(jnp.float32)


_bits_call = pl.pallas_call(
    _bits_body,
    out_shape=jax.ShapeDtypeStruct((NB, G), jnp.float32),
)


def _stage_idx(idx_hbm, idx_v, base, is_last):
    @pl.when(jnp.logical_not(is_last))
    def _():
        pltpu.sync_copy(idx_hbm.at[pl.ds(base, BPW)], idx_v)

    @pl.when(is_last)
    def _():
        pltpu.sync_copy(
            idx_hbm.at[pl.ds(base, BPW_LAST)], idx_v.at[pl.ds(0, BPW_LAST)]
        )
        for t in range(BPW_LAST, BPW, L):
            idx_v[pl.ds(t, L)] = jnp.zeros((L,), jnp.int32)


@functools.cache
def _make_sc_expand():
    mesh = plsc.VectorSubcoreMesh(
        core_axis_name="c", subcore_axis_name="s", num_cores=NC, num_subcores=NS
    )
    return pl.kernel(
        _sc_expand_body,
        out_type=[
            jax.ShapeDtypeStruct((N, D), jnp.float32),
            jax.ShapeDtypeStruct((NW, G), jnp.int32),
        ],
        mesh=mesh,
        compiler_params=pltpu.CompilerParams(needs_layout_passes=False),
        scratch_types=[
            pltpu.VMEM((BPW,), jnp.int32),
            pltpu.VMEM((CH, D), jnp.float32),
            pltpu.VMEM((CH, D), jnp.float32),
            pltpu.VMEM((CH, D), jnp.float32),
            pltpu.VMEM((G,), jnp.int32),
            pltpu.VMEM_SHARED((G, D), jnp.float32),
            pltpu.SemaphoreType.DMA,
            pltpu.SemaphoreType.DMA,
            pltpu.SemaphoreType.DMA,
            pltpu.SemaphoreType.DMA,
            pltpu.SemaphoreType.DMA,
            pltpu.SemaphoreType.DMA,
        ],
    )


def _sc_expand_body(emb_hbm, idx_hbm, out_hbm, parts_hbm,
                    idx_v, rows0, rows1, rows2, cnt_v, semb,
                    gs0, gs1, gs2, ss0, ss1, ss2):
    wid = lax.axis_index("s") * NC + lax.axis_index("c")
    sid = lax.axis_index("s")
    base = wid * BPW
    is_last = wid == NW - 1
    _stage_idx(idx_hbm, idx_v, base, is_last)

    GPT = G // NS
    pltpu.sync_copy(
        emb_hbm.at[pl.ds(sid * GPT, GPT)], semb.at[pl.ds(sid * GPT, GPT)]
    )
    plsc.subcore_barrier()

    bufs = (rows0, rows1, rows2)
    gsems = (gs0, gs1, gs2)
    ssems = (ss0, ss1, ss2)
    NBUF = 3
    FULL = BPW_LAST // CH

    def start_gather(c):
        b = c % NBUF
        return pltpu.async_copy(
            semb.at[idx_v.at[pl.ds(c * CH, CH)]], bufs[b], gsems[b]
        )

    g = [start_gather(0), start_gather(1), None]
    s = [None, None, None]

    def zero_body(i, carry):
        cnt_v[pl.ds(i * L, L)] = jnp.zeros((L,), jnp.int32)
        return carry

    lax.fori_loop(0, G // L, zero_body, 0)

    iota16 = lax.iota(jnp.int32, L)
    ones16 = jnp.ones((L,), jnp.int32)

    def count_body(j, carry):
        v = idx_v[pl.ds(j * L, L)]
        valid = (base + j * L + iota16) < N
        plsc.addupdate_scatter(cnt_v, [v], ones16, mask=valid)
        return carry

    lax.fori_loop(0, BPW // L, count_body, 0)
    pltpu.sync_copy(cnt_v, parts_hbm.at[wid])

    for c in range(NCH):
        b = c % NBUF
        g[b].wait()
        if c < FULL:
            s[b] = pltpu.async_copy(
                bufs[b], out_hbm.at[pl.ds(base + c * CH, CH)], ssems[b]
            )
        else:
            def store_full(b=b, c=c):
                pltpu.sync_copy(bufs[b], out_hbm.at[pl.ds(base + c * CH, CH)])

            if c == PARTIAL_C:
                pl.when(jnp.logical_not(is_last))(store_full)

                @pl.when(is_last)
                def _(b=b, c=c):
                    pltpu.sync_copy(
                        bufs[b].at[pl.ds(0, PARTIAL_LEN)],
                        out_hbm.at[pl.ds(base + c * CH, PARTIAL_LEN)],
                    )
            else:
                pl.when(jnp.logical_not(is_last))(store_full)
            s[b] = None

        n = c + 2
        if n < NCH:
            nb = n % NBUF
            if s[nb] is not None:
                s[nb].wait()
                s[nb] = None
            g[nb] = start_gather(n)

    for b in range(NBUF):
        if s[b] is not None:
            s[b].wait()


def kernel(tau, batch):
    emb, alpha, sigma = _emb_call(tau.reshape(G, 1))
    out, parts = _make_sc_expand()(emb, batch)
    bits = _bits_call(parts)
    return out, alpha, sigma, bits.T


# speedup vs baseline: 1.0133x; 1.0133x over previous
import functools

import jax
import jax.numpy as jnp
from jax import lax
from jax.experimental import pallas as pl
from jax.experimental.pallas import tpu as pltpu
from jax.experimental.pallas import tpu_sc as plsc

G = 4096
D = 128
N = 100000
NB = 8
NC, NS, L = 2, 16, 16
NW = NC * NS
BPW = 3136
CH = 224
NCH = BPW // CH
BPW_LAST = N - (NW - 1) * BPW
PARTIAL_C = BPW_LAST // CH
PARTIAL_LEN = BPW_LAST - PARTIAL_C * CH


def _emb_body(tau_ref, emb_ref, alpha_ref, sigma_ref):
    tau = tau_ref[:, :]
    col = lax.broadcasted_iota(jnp.int32, (1, D), 1)
    half = col >= D // 2
    j = jnp.where(half, col - D // 2, col).astype(jnp.float32)
    freq = jnp.exp(-jnp.log(10000.0) * j / (D // 2))
    phase = jnp.where(half, jnp.float32(jnp.pi / 2), jnp.float32(0.0))
    x = tau * freq + phase
    x2 = x * x
    p = jnp.float32(-1.0 / 39916800.0)
    p = p * x2 + jnp.float32(1.0 / 362880.0)
    p = p * x2 + jnp.float32(-1.0 / 5040.0)
    p = p * x2 + jnp.float32(1.0 / 120.0)
    p = p * x2 + jnp.float32(-1.0 / 6.0)
    emb_ref[:, :] = x * (p * x2 + jnp.float32(1.0))
    alpha_ref[:, :] = 1.0 - tau
    sigma_ref[:, :] = tau


_emb_call = pl.pallas_call(
    _emb_body,
    out_shape=[
        jax.ShapeDtypeStruct((G, D), jnp.float32),
        jax.ShapeDtypeStruct((G, 1), jnp.float32),
        jax.ShapeDtypeStruct((G, 1), jnp.float32),
    ],
)


def _bits_body(parts_ref, bits_ref):
    counts = jnp.sum(parts_ref[:, :], axis=0, keepdims=True)
    row = lax.broadcasted_iota(jnp.int32, (NB, G), 0)
    bits_ref[:, :] = ((counts >> row) & 1).astype(jnp.float32)


_bits_call = pl.pallas_call(
    _bits_body,
    out_shape=jax.ShapeDtypeStruct((NB, G), jnp.float32),
)


def _stage_idx(idx_hbm, idx_v, base, is_last):
    @pl.when(jnp.logical_not(is_last))
    def _():
        pltpu.sync_copy(idx_hbm.at[pl.ds(base, BPW)], idx_v)

    @pl.when(is_last)
    def _():
        pltpu.sync_copy(
            idx_hbm.at[pl.ds(base, BPW_LAST)], idx_v.at[pl.ds(0, BPW_LAST)]
        )
        for t in range(BPW_LAST, BPW, L):
            idx_v[pl.ds(t, L)] = jnp.zeros((L,), jnp.int32)


@functools.cache
def _make_sc_expand():
    mesh = plsc.VectorSubcoreMesh(
        core_axis_name="c", subcore_axis_name="s", num_cores=NC, num_subcores=NS
    )
    return pl.kernel(
        _sc_expand_body,
        out_type=[
            jax.ShapeDtypeStruct((N, D), jnp.float32),
            jax.ShapeDtypeStruct((NW, G), jnp.int32),
        ],
        mesh=mesh,
        compiler_params=pltpu.CompilerParams(needs_layout_passes=False),
        scratch_types=[
            pltpu.VMEM((BPW,), jnp.int32),
            pltpu.VMEM((CH, D), jnp.float32),
            pltpu.VMEM((CH, D), jnp.float32),
            pltpu.VMEM((CH, D), jnp.float32),
            pltpu.VMEM((G,), jnp.int32),
            pltpu.VMEM_SHARED((G, D), jnp.float32),
            pltpu.SemaphoreType.DMA,
            pltpu.SemaphoreType.DMA,
            pltpu.SemaphoreType.DMA,
            pltpu.SemaphoreType.DMA,
            pltpu.SemaphoreType.DMA,
            pltpu.SemaphoreType.DMA,
        ],
    )


def _sc_expand_body(emb_hbm, idx_hbm, out_hbm, parts_hbm,
                    idx_v, rows0, rows1, rows2, cnt_v, semb,
                    gs0, gs1, gs2, ss0, ss1, ss2):
    wid = lax.axis_index("s") * NC + lax.axis_index("c")
    sid = lax.axis_index("s")
    base = wid * BPW
    is_last = wid == NW - 1
    _stage_idx(idx_hbm, idx_v, base, is_last)

    GPT = G // NS
    pltpu.sync_copy(
        emb_hbm.at[pl.ds(sid * GPT, GPT)], semb.at[pl.ds(sid * GPT, GPT)]
    )
    plsc.subcore_barrier()

    bufs = (rows0, rows1, rows2)
    gsems = (gs0, gs1, gs2)
    ssems = (ss0, ss1, ss2)
    NBUF = 3
    FULL = BPW_LAST // CH

    def start_gather(c):
        b = c % NBUF
        return pltpu.async_copy(
            semb.at[idx_v.at[pl.ds(c * CH, CH)]], bufs[b], gsems[b]
        )

    g = [start_gather(0), start_gather(1), None]
    s = [None, None, None]

    def zero_body(i, carry):
        for u in range(4):
            cnt_v[pl.ds((i * 4 + u) * L, L)] = jnp.zeros((L,), jnp.int32)
        return carry

    lax.fori_loop(0, G // L // 4, zero_body, 0)

    iota16 = lax.iota(jnp.int32, L)
    ones16 = jnp.ones((L,), jnp.int32)

    def count_body(j, carry):
        for u in range(2):
            jj = j * 2 + u
            v = idx_v[pl.ds(jj * L, L)]
            valid = (base + jj * L + iota16) < N
            plsc.addupdate_scatter(cnt_v, [v], ones16, mask=valid)
        return carry

    lax.fori_loop(0, BPW // L // 2, count_body, 0)
    pltpu.sync_copy(cnt_v, parts_hbm.at[wid])

    for c in range(NCH):
        b = c % NBUF
        g[b].wait()
        if c < FULL:
            s[b] = pltpu.async_copy(
                bufs[b], out_hbm.at[pl.ds(base + c * CH, CH)], ssems[b]
            )
        else:
            def store_full(b=b, c=c):
                pltpu.sync_copy(bufs[b], out_hbm.at[pl.ds(base + c * CH, CH)])

            if c == PARTIAL_C:
                pl.when(jnp.logical_not(is_last))(store_full)

                @pl.when(is_last)
                def _(b=b, c=c):
                    pltpu.sync_copy(
                        bufs[b].at[pl.ds(0, PARTIAL_LEN)],
                        out_hbm.at[pl.ds(base + c * CH, PARTIAL_LEN)],
                    )
            else:
                pl.when(jnp.logical_not(is_last))(store_full)
            s[b] = None

        n = c + 2
        if n < NCH:
            nb = n % NBUF
            if s[nb] is not None:
                s[nb].wait()
                s[nb] = None
            g[nb] = start_gather(n)

    for b in range(NBUF):
        if s[b] is not None:
            s[b].wait()


def kernel(tau, batch):
    emb, alpha, sigma = _emb_call(tau.reshape(G, 1))
    out, parts = _make_sc_expand()(emb, batch)
    bits = _bits_call(parts)
    return out, alpha, sigma, bits.T
